# baseline (speedup 1.0000x reference)
import functools

import jax
import jax.numpy as jnp
from jax import lax
from jax.experimental import pallas as pl
from jax.experimental.pallas import tpu as pltpu
from jax.experimental.pallas import tpu_sc as plsc

B = 16384
D = 128
L = 16
NSUB = D // L

NC = 2
NS = 16
NW = NC * NS
BPW = B // NW

CH = 128
NCH = BPW // CH
HH = CH // 2


def _transh_body(head_hbm, rel_hbm, tail_hbm, ent_hbm, rele_hbm, relh_hbm,
                 out_hbm, hidx, tidx, ridx, hbuf, tbuf, wbuf, rbuf, obuf, sem):
    cid = lax.axis_index("c")
    sid = lax.axis_index("s")
    wid = sid * NC + cid
    base = wid * BPW

    pltpu.sync_copy(head_hbm.at[pl.ds(base, BPW)], hidx)
    pltpu.sync_copy(tail_hbm.at[pl.ds(base, BPW)], tidx)
    pltpu.sync_copy(rel_hbm.at[pl.ds(base, BPW)], ridx)

    def row(i, rcarry):
        acc = jnp.zeros((L,), jnp.float32)
        hmts = []
        ws = []
        for j in range(NSUB):
            csl = pl.ds(j * L, L)
            h = hbuf[i, csl]
            t = tbuf[i, csl]
            w = wbuf[i, csl]
            hmt = h - t
            acc = acc + hmt * w
            hmts.append(hmt)
            ws.append(w)
        d = jnp.sum(acc)
        for j in range(NSUB):
            csl = pl.ds(j * L, L)
            r = rbuf[i, csl]
            obuf[i, csl] = hmts[j] + r - ws[j] * d
        return rcarry

    def chunk(c, carry):
        cps = [None, None]
        for hf in (0, 1):
            isl = pl.ds(c * CH + hf * HH, HH)
            dsl = pl.ds(hf * HH, HH)
            cps[hf] = (
                pltpu.async_copy(ent_hbm.at[hidx.at[isl]],
                                 hbuf.at[dsl], sem.at[hf]),
                pltpu.async_copy(ent_hbm.at[tidx.at[isl]],
                                 tbuf.at[dsl], sem.at[hf]),
                pltpu.async_copy(relh_hbm.at[ridx.at[isl]],
                                 wbuf.at[dsl], sem.at[hf]),
                pltpu.async_copy(rele_hbm.at[ridx.at[isl]],
                                 rbuf.at[dsl], sem.at[hf]),
            )
        for hf in (0, 1):
            for cp in cps[hf]:
                cp.wait()
            lax.fori_loop(hf * HH, (hf + 1) * HH, row, 0)
        pltpu.sync_copy(obuf, out_hbm.at[pl.ds(base + c * CH, CH)])
        return carry

    lax.fori_loop(0, NCH, chunk, 0)


_transh = functools.partial(
    pl.kernel,
    out_type=jax.ShapeDtypeStruct((B, D), jnp.float32),
    mesh=plsc.VectorSubcoreMesh(core_axis_name="c", subcore_axis_name="s"),
    compiler_params=pltpu.CompilerParams(needs_layout_passes=False),
    scratch_types=[
        pltpu.VMEM((BPW,), jnp.int32),
        pltpu.VMEM((BPW,), jnp.int32),
        pltpu.VMEM((BPW,), jnp.int32),
        pltpu.VMEM((CH, D), jnp.float32),
        pltpu.VMEM((CH, D), jnp.float32),
        pltpu.VMEM((CH, D), jnp.float32),
        pltpu.VMEM((CH, D), jnp.float32),
        pltpu.VMEM((CH, D), jnp.float32),
        pltpu.SemaphoreType.DMA((2,)),

    ],
)(_transh_body)


def kernel(head, relation, tail, ent_emb, rel_emb, rel_hyper):
    return _transh(head, relation, tail, ent_emb, rel_emb, rel_hyper)
